# Initial kernel scaffold; baseline (speedup 1.0000x reference)
import jax
import jax.numpy as jnp
from jax.experimental import pallas as pl


def kernel(x, L, W, b, fc_w, fc_b, y):
    raise NotImplementedError("write your pallas kernel here")


# speedup vs baseline: 2.0662x; 2.0662x over previous
import functools

import jax
import jax.numpy as jnp
from jax.experimental import pallas as pl
from jax.experimental.pallas import tpu as pltpu


def _body(x_ref, m_ref, w_ref, bias_ref, fc_ref, out_ref, p_ref, q_ref, *,
          kk, filt, nfp, ng, tb):
    gf = filt // ng

    @pl.when(pl.program_id(0) == 0)
    def _build_q():
        mv = m_ref[...]
        r = jax.lax.broadcasted_iota(jnp.int32, (nfp, nfp), 0)
        c = jax.lax.broadcasted_iota(jnp.int32, (nfp, nfp), 1)
        p0 = (r == c).astype(jnp.float32)
        p_ref[0, :, :] = p0
        p_ref[1, :, :] = mv
        t0, t1 = p0, mv
        for k in range(2, kk):
            t2 = 2.0 * jax.lax.dot(mv, t1, precision=jax.lax.Precision.HIGHEST,
                                   preferred_element_type=jnp.float32) - t0
            p_ref[k, :, :] = t2
            t0, t1 = t1, t2

        def qbody(f, carry):
            acc = p_ref[0, :, :] * w_ref[0, f]
            for k in range(1, kk):
                acc = acc + p_ref[k, :, :] * w_ref[k, f]
            q_ref[f, :, :] = acc.astype(jnp.bfloat16)
            return carry

        jax.lax.fori_loop(0, filt, qbody, 0)

    xb = x_ref[...].astype(jnp.bfloat16)

    def gbody(g, acc):
        qg = jnp.concatenate([q_ref[g * gf + j, :, :] for j in range(gf)],
                             axis=1)
        gv = jax.lax.dot(xb, qg, preferred_element_type=jnp.float32)
        gv = jnp.maximum(gv + bias_ref[g, :, :], 0.0).astype(jnp.bfloat16)
        return acc + jax.lax.dot(gv, fc_ref[g, :, :],
                                 preferred_element_type=jnp.float32)

    acc = jax.lax.fori_loop(
        0, ng, gbody, jnp.zeros((tb, 128), jnp.float32))
    out_ref[...] = acc


def kernel(x, L, W, b, fc_w, fc_b, y):
    B, N, F = x.shape
    K, FILT = W.shape
    C = fc_w.shape[1]
    NF = N * F
    NFP = 384
    TB = 256
    NG = 8
    GF = FILT // NG
    GW = GF * NFP

    x2 = jnp.pad(x.reshape(B, NF), ((0, 0), (0, NFP - NF)))
    M = jnp.kron(L.T, jnp.eye(F, dtype=L.dtype))
    Mp = jnp.pad(M, ((0, NFP - NF), (0, NFP - NF)))
    bias_g = jnp.repeat(b.reshape(FILT), NFP).reshape(NG, 1, GW)
    fc3 = jnp.pad(fc_w.reshape(FILT, NF, C),
                  ((0, 0), (0, NFP - NF), (0, 128 - C)))
    fcp = fc3.reshape(NG, GW, 128).astype(jnp.bfloat16)

    body = functools.partial(_body, kk=K, filt=FILT, nfp=NFP, ng=NG, tb=TB)
    out = pl.pallas_call(
        body,
        grid=(B // TB,),
        in_specs=[
            pl.BlockSpec((TB, NFP), lambda i: (i, 0)),
            pl.BlockSpec((NFP, NFP), lambda i: (0, 0)),
            pl.BlockSpec(memory_space=pltpu.SMEM),
            pl.BlockSpec((NG, 1, GW), lambda i: (0, 0, 0)),
            pl.BlockSpec((NG, GW, 128), lambda i: (0, 0, 0)),
        ],
        out_specs=pl.BlockSpec((TB, 128), lambda i: (i, 0)),
        out_shape=jax.ShapeDtypeStruct((B, 128), jnp.float32),
        scratch_shapes=[
            pltpu.VMEM((K, NFP, NFP), jnp.float32),
            pltpu.VMEM((FILT, NFP, NFP), jnp.bfloat16),
        ],
        compiler_params=pltpu.CompilerParams(
            dimension_semantics=("arbitrary",)),
    )(x2, Mp, W, bias_g, fcp)
    return out[:, :C] + fc_b[None, :]


# speedup vs baseline: 2.2660x; 1.0967x over previous
import functools

import jax
import jax.numpy as jnp
from jax.experimental import pallas as pl
from jax.experimental.pallas import tpu as pltpu


def _body(x_ref, m_ref, w_ref, bias_ref, fc_ref, out_ref, p_ref, q_ref, *,
          kk, filt, nfp, ng, tb):
    gf = filt // ng

    @pl.when(pl.program_id(0) == 0)
    def _build_q():
        mv = m_ref[...]
        r = jax.lax.broadcasted_iota(jnp.int32, (nfp, nfp), 0)
        c = jax.lax.broadcasted_iota(jnp.int32, (nfp, nfp), 1)
        p0 = (r == c).astype(jnp.float32)
        p_ref[0, :, :] = p0
        p_ref[1, :, :] = mv
        t0, t1 = p0, mv
        for k in range(2, kk):
            t2 = 2.0 * jax.lax.dot(mv, t1, precision=jax.lax.Precision.HIGHEST,
                                   preferred_element_type=jnp.float32) - t0
            p_ref[k, :, :] = t2
            t0, t1 = t1, t2

        def qbody(f, carry):
            acc = p_ref[0, :, :] * w_ref[0, f]
            for k in range(1, kk):
                acc = acc + p_ref[k, :, :] * w_ref[k, f]
            q_ref[f, :, :] = acc.astype(jnp.bfloat16)
            return carry

        jax.lax.fori_loop(0, filt, qbody, 0)

    xb = x_ref[...].astype(jnp.bfloat16)

    def gbody(g, acc):
        qg = jnp.concatenate([q_ref[g * gf + j, :, :] for j in range(gf)],
                             axis=1)
        gv = jax.lax.dot(xb, qg, preferred_element_type=jnp.float32)
        gv = jnp.maximum(gv + bias_ref[g, :, :], 0.0).astype(jnp.bfloat16)
        return acc + jax.lax.dot(gv, fc_ref[g, :, :],
                                 preferred_element_type=jnp.float32)

    acc = jax.lax.fori_loop(
        0, ng, gbody, jnp.zeros((tb, 128), jnp.float32))
    out_ref[...] = acc


def kernel(x, L, W, b, fc_w, fc_b, y):
    B, N, F = x.shape
    K, FILT = W.shape
    C = fc_w.shape[1]
    NF = N * F
    NFP = 384
    TB = 512
    NG = 8
    GF = FILT // NG
    GW = GF * NFP

    x2 = jnp.pad(x.reshape(B, NF), ((0, 0), (0, NFP - NF)))
    M = jnp.kron(L.T, jnp.eye(F, dtype=L.dtype))
    Mp = jnp.pad(M, ((0, NFP - NF), (0, NFP - NF)))
    bias_g = jnp.repeat(b.reshape(FILT), NFP).reshape(NG, 1, GW)
    fc3 = jnp.pad(fc_w.reshape(FILT, NF, C),
                  ((0, 0), (0, NFP - NF), (0, 128 - C)))
    fcp = fc3.reshape(NG, GW, 128).astype(jnp.bfloat16)

    body = functools.partial(_body, kk=K, filt=FILT, nfp=NFP, ng=NG, tb=TB)
    out = pl.pallas_call(
        body,
        grid=(B // TB,),
        in_specs=[
            pl.BlockSpec((TB, NFP), lambda i: (i, 0)),
            pl.BlockSpec((NFP, NFP), lambda i: (0, 0)),
            pl.BlockSpec(memory_space=pltpu.SMEM),
            pl.BlockSpec((NG, 1, GW), lambda i: (0, 0, 0)),
            pl.BlockSpec((NG, GW, 128), lambda i: (0, 0, 0)),
        ],
        out_specs=pl.BlockSpec((TB, 128), lambda i: (i, 0)),
        out_shape=jax.ShapeDtypeStruct((B, 128), jnp.float32),
        scratch_shapes=[
            pltpu.VMEM((K, NFP, NFP), jnp.float32),
            pltpu.VMEM((FILT, NFP, NFP), jnp.bfloat16),
        ],
        compiler_params=pltpu.CompilerParams(
            dimension_semantics=("arbitrary",)),
    )(x2, Mp, W, bias_g, fcp)
    return out[:, :C] + fc_b[None, :]


# speedup vs baseline: 2.2674x; 1.0007x over previous
import functools

import jax
import jax.numpy as jnp
from jax.experimental import pallas as pl
from jax.experimental.pallas import tpu as pltpu


def _body(x_ref, m_ref, w_ref, bias_ref, fc_ref, out_ref, p_ref, q_ref, *,
          kk, filt, nfp, ng, tb):
    gf = filt // ng

    @pl.when(pl.program_id(0) == 0)
    def _build_q():
        mv = m_ref[...]
        r = jax.lax.broadcasted_iota(jnp.int32, (nfp, nfp), 0)
        c = jax.lax.broadcasted_iota(jnp.int32, (nfp, nfp), 1)
        p0 = (r == c).astype(jnp.float32)
        p_ref[0, :, :] = p0
        p_ref[1, :, :] = mv
        t0, t1 = p0, mv
        for k in range(2, kk):
            t2 = 2.0 * jax.lax.dot(mv, t1, precision=jax.lax.Precision.HIGHEST,
                                   preferred_element_type=jnp.float32) - t0
            p_ref[k, :, :] = t2
            t0, t1 = t1, t2

        def qbody(g, carry):
            for j in range(gf):
                f = g * gf + j
                acc = p_ref[0, :, :] * w_ref[0, f]
                for k in range(1, kk):
                    acc = acc + p_ref[k, :, :] * w_ref[k, f]
                q_ref[g, :, j * nfp:(j + 1) * nfp] = acc.astype(jnp.bfloat16)
            return carry

        jax.lax.fori_loop(0, ng, qbody, 0)

    xb = x_ref[...].astype(jnp.bfloat16)

    def gbody(g, acc):
        gv = jax.lax.dot(xb, q_ref[g, :, :],
                         preferred_element_type=jnp.float32)
        gv = jnp.maximum(gv + bias_ref[g, :, :], 0.0).astype(jnp.bfloat16)
        return acc + jax.lax.dot(gv, fc_ref[g, :, :],
                                 preferred_element_type=jnp.float32)

    acc = jax.lax.fori_loop(
        0, ng, gbody, jnp.zeros((tb, 128), jnp.float32))
    out_ref[...] = acc


def kernel(x, L, W, b, fc_w, fc_b, y):
    B, N, F = x.shape
    K, FILT = W.shape
    C = fc_w.shape[1]
    NF = N * F
    NFP = 384
    TB = 512
    NG = 8
    GF = FILT // NG
    GW = GF * NFP

    x2 = jnp.pad(x.reshape(B, NF), ((0, 0), (0, NFP - NF)))
    M = jnp.kron(L.T, jnp.eye(F, dtype=L.dtype))
    Mp = jnp.pad(M, ((0, NFP - NF), (0, NFP - NF)))
    bias_g = jnp.repeat(b.reshape(FILT), NFP).reshape(NG, 1, GW)
    fc3 = jnp.pad(fc_w.reshape(FILT, NF, C),
                  ((0, 0), (0, NFP - NF), (0, 128 - C)))
    fcp = fc3.reshape(NG, GW, 128).astype(jnp.bfloat16)

    body = functools.partial(_body, kk=K, filt=FILT, nfp=NFP, ng=NG, tb=TB)
    out = pl.pallas_call(
        body,
        grid=(B // TB,),
        in_specs=[
            pl.BlockSpec((TB, NFP), lambda i: (i, 0)),
            pl.BlockSpec((NFP, NFP), lambda i: (0, 0)),
            pl.BlockSpec(memory_space=pltpu.SMEM),
            pl.BlockSpec((NG, 1, GW), lambda i: (0, 0, 0)),
            pl.BlockSpec((NG, GW, 128), lambda i: (0, 0, 0)),
        ],
        out_specs=pl.BlockSpec((TB, 128), lambda i: (i, 0)),
        out_shape=jax.ShapeDtypeStruct((B, 128), jnp.float32),
        scratch_shapes=[
            pltpu.VMEM((K, NFP, NFP), jnp.float32),
            pltpu.VMEM((NG, NFP, GW), jnp.bfloat16),
        ],
        compiler_params=pltpu.CompilerParams(
            dimension_semantics=("arbitrary",)),
    )(x2, Mp, W, bias_g, fcp)
    return out[:, :C] + fc_b[None, :]
